# baseline (speedup 1.0000x reference)
import functools

import jax
import jax.numpy as jnp
from jax import lax
from jax.experimental import pallas as pl
from jax.experimental.pallas import tpu as pltpu
from jax.experimental.pallas import tpu_sc as plsc

NUM_ROWS = 1000000
DIM = 32
B_TOKENS = 16384
SEQ = 26
B_FLAT = B_TOKENS * SEQ

_info = plsc.get_sparse_core_info()
NC = _info.num_cores
NS = _info.num_subcores
NW = NC * NS
B_PER_W = B_FLAT // NW

TN = 2048
GRID = (NUM_ROWS + TN - 1) // TN


def _tr_body(in_ref, out_ref):
    v = in_ref[...]
    t = v.T
    t4 = t.reshape(TN // 4, 4, DIM)
    out_ref[...] = jnp.concatenate(
        [t4[:, 0, :], t4[:, 1, :], t4[:, 2, :], t4[:, 3, :]], axis=1)


_transpose = pl.pallas_call(
    _tr_body,
    grid=(GRID,),
    in_specs=[pl.BlockSpec((DIM, TN), lambda g: (0, g))],
    out_specs=pl.BlockSpec((TN // 4, DIM * 4), lambda g: (g, 0)),
    out_shape=jax.ShapeDtypeStruct((NUM_ROWS // 4, DIM * 4), jnp.float32),
)

C4 = 256
NCH = B_PER_W // C4
NBUF = 2

_mesh = plsc.VectorSubcoreMesh(core_axis_name="c", subcore_axis_name="s")


@functools.partial(
    pl.kernel,
    mesh=_mesh,
    out_type=jax.ShapeDtypeStruct((B_FLAT, DIM), jnp.float32),
    compiler_params=pltpu.CompilerParams(
        use_tc_tiling_on_sc=False, needs_layout_passes=False),
    scratch_types=[
        pltpu.VMEM((B_PER_W,), jnp.int32),
        pltpu.VMEM((B_PER_W,), jnp.int32),
        [pltpu.VMEM((C4, DIM * 4), jnp.float32) for _ in range(NBUF)],
        [pltpu.VMEM((C4, DIM), jnp.float32) for _ in range(NBUF)],
        pltpu.SemaphoreType.DMA,
        pltpu.SemaphoreType.DMA,
    ],
)
def _gather4(idx_hbm, table_hbm, out_hbm, idx_v, ridx_v, rows, outs,
             gsem, osem):
    wid = lax.axis_index("s") * NC + lax.axis_index("c")
    base = wid * B_PER_W

    pltpu.sync_copy(idx_hbm.at[pl.ds(base, B_PER_W)], idx_v)

    def rbody(m, carry):
        v = idx_v[pl.ds(m * 16, 16)]
        ridx_v[pl.ds(m * 16, 16)] = lax.shift_right_logical(v, 2)
        return carry

    lax.fori_loop(0, B_PER_W // 16, rbody, 0)

    def start_gather(c, b):
        pltpu.async_copy(
            table_hbm.at[ridx_v.at[pl.ds(c * C4, C4)]], rows[b], gsem)

    def start_store(c, b):
        pltpu.async_copy(
            outs[b], out_hbm.at[pl.ds(base + c * C4, C4)], osem)

    def drain_gather(b):
        pltpu.make_async_copy(
            table_hbm.at[ridx_v.at[pl.ds(0, C4)]], rows[b], gsem).wait()

    def drain_store(b):
        pltpu.make_async_copy(
            outs[b], out_hbm.at[pl.ds(base, C4)], osem).wait()

    iota16 = lax.iota(jnp.int32, 16)

    def select(coff, b):
        rbuf = rows[b]
        obuf = outs[b]

        def mbody(m, carry):
            r16 = iota16 + m * 16
            q16 = (idx_v[pl.ds(coff + m * 16, 16)] & 3) * DIM
            for k in range(DIM):
                vals = plsc.load_gather(rbuf, [r16, q16 + k])
                plsc.store_scatter(obuf, [r16, jnp.full((16,), k, jnp.int32)],
                                   vals)
            return carry

        lax.fori_loop(0, C4 // 16, mbody, 0)

    G = NCH // 2

    start_gather(0, 0)

    def gbody(g, carry):
        c0 = 2 * g
        start_gather(c0 + 1, 1)
        drain_gather(0)

        @pl.when(g >= 1)
        def _():
            drain_store(0)

        select(c0 * C4, 0)
        start_store(c0, 0)

        @pl.when(g < G - 1)
        def _():
            start_gather(c0 + 2, 0)

        drain_gather(1)

        @pl.when(g >= 1)
        def _():
            drain_store(1)

        select((c0 + 1) * C4, 1)
        start_store(c0 + 1, 1)
        return carry

    lax.fori_loop(0, G, gbody, 0)
    drain_store(0)
    drain_store(1)


def kernel(x, mat):
    xf = x.reshape(B_FLAT)
    mat4 = _transpose(mat.T)
    out = _gather4(xf, mat4)
    return out.reshape(B_TOKENS, SEQ, DIM)


# speedup vs baseline: 1.5780x; 1.5780x over previous
import functools

import jax
import jax.numpy as jnp
from jax import lax
from jax.experimental import pallas as pl
from jax.experimental.pallas import tpu as pltpu
from jax.experimental.pallas import tpu_sc as plsc

NUM_ROWS = 1000000
DIM = 32
B_TOKENS = 16384
SEQ = 26
B_FLAT = B_TOKENS * SEQ

_info = plsc.get_sparse_core_info()
NC = _info.num_cores
NS = _info.num_subcores
NW = NC * NS
B_PER_W = B_FLAT // NW
CHUNK = 1024
N_CHUNKS = B_PER_W // CHUNK
NBUF = 3

_mesh = plsc.VectorSubcoreMesh(core_axis_name="c", subcore_axis_name="s")


@functools.partial(
    pl.kernel,
    mesh=_mesh,
    out_type=jax.ShapeDtypeStruct((B_FLAT, DIM), jnp.float32),
    compiler_params=pltpu.CompilerParams(use_tc_tiling_on_sc=False),
    scratch_types=[
        pltpu.VMEM((B_PER_W,), jnp.int32),
        [pltpu.VMEM((CHUNK, DIM), jnp.float32) for _ in range(NBUF)],
        pltpu.SemaphoreType.DMA,
        pltpu.SemaphoreType.DMA,
    ],
)
def _gather(idx_hbm, table_hbm, out_hbm, idx_v, rows, gsem, osem):
    wid = lax.axis_index("s") * NC + lax.axis_index("c")
    base = wid * B_PER_W

    pltpu.sync_copy(idx_hbm.at[pl.ds(base, B_PER_W)], idx_v)

    def start_gather(i):
        return pltpu.async_copy(
            table_hbm.at[idx_v.at[pl.ds(i * CHUNK, CHUNK)]],
            rows[i % NBUF], gsem)

    def start_store(i):
        return pltpu.async_copy(
            rows[i % NBUF], out_hbm.at[pl.ds(base + i * CHUNK, CHUNK)], osem)

    gathers = [start_gather(0), start_gather(1)]
    stores = []
    for i in range(N_CHUNKS):
        if i + 2 < N_CHUNKS:
            if i >= 1:
                stores[i - 1].wait()
            gathers.append(start_gather(i + 2))
        gathers[i].wait()
        stores.append(start_store(i))
    stores[N_CHUNKS - 3].wait()
    stores[N_CHUNKS - 2].wait()
    stores[N_CHUNKS - 1].wait()


def kernel(x, mat):
    xf = x.reshape(B_FLAT)
    out = _gather(xf, mat)
    return out.reshape(B_TOKENS, SEQ, DIM)


# speedup vs baseline: 1.9835x; 1.2569x over previous
import functools

import jax
import jax.numpy as jnp
from jax import lax
from jax.experimental import pallas as pl
from jax.experimental.pallas import tpu as pltpu
from jax.experimental.pallas import tpu_sc as plsc

NUM_ROWS = 1000000
DIM = 32
B_TOKENS = 16384
SEQ = 26
B_FLAT = B_TOKENS * SEQ
OUT_ROWS = B_TOKENS * 32 * 4

_info = plsc.get_sparse_core_info()
NC = _info.num_cores
NS = _info.num_subcores
NW = NC * NS
B_PER_W = B_FLAT // NW
CHUNK = 1024
N_CHUNKS = B_PER_W // CHUNK
NBUF = 2
DIV26_MUL = 80660
DIV26_SHIFT = 21

_mesh = plsc.VectorSubcoreMesh(core_axis_name="c", subcore_axis_name="s")


@functools.partial(
    pl.kernel,
    mesh=_mesh,
    out_type=jax.ShapeDtypeStruct((OUT_ROWS, DIM), jnp.float32),
    compiler_params=pltpu.CompilerParams(use_tc_tiling_on_sc=False),
    scratch_types=[
        pltpu.VMEM((B_PER_W,), jnp.int32),
        pltpu.VMEM((N_CHUNKS, CHUNK), jnp.int32),
        [pltpu.VMEM((CHUNK, DIM), jnp.float32) for _ in range(NBUF)],
        pltpu.SemaphoreType.DMA,
        pltpu.SemaphoreType.DMA,
    ],
)
def _gather(idx_hbm, table_hbm, out_hbm, idx_v, pos_v, rows, gsem, osem):
    wid = lax.axis_index("s") * NC + lax.axis_index("c")
    base = wid * B_PER_W
    tbase = wid * (B_PER_W // SEQ)

    pltpu.sync_copy(idx_hbm.at[pl.ds(base, B_PER_W)], idx_v)

    iota16 = lax.iota(jnp.int32, 16)

    def pbody(mm, carry):
        c = mm // (CHUNK // 16)
        m = mm % (CHUNK // 16)
        jl = iota16 + (c * CHUNK + m * 16)
        t = lax.shift_right_logical(jl * DIV26_MUL, DIV26_SHIFT)
        s = jl - t * SEQ
        pos_v[c, pl.ds(m * 16, 16)] = ((tbase + t) * 32 + s) * 4
        return carry

    lax.fori_loop(0, N_CHUNKS * (CHUNK // 16), pbody, 0)

    def start_gather(c):
        return pltpu.async_copy(
            table_hbm.at[idx_v.at[pl.ds(c * CHUNK, CHUNK)]],
            rows[c % NBUF], gsem)

    def start_store(c):
        return pltpu.async_copy(
            rows[c % NBUF], out_hbm.at[pos_v.at[c]], osem)

    gathers = [start_gather(0)]
    stores = []
    for c in range(N_CHUNKS):
        if c + 1 < N_CHUNKS:
            if c >= 1:
                stores[c - 1].wait()
            gathers.append(start_gather(c + 1))
        gathers[c].wait()
        stores.append(start_store(c))
    stores[N_CHUNKS - 2].wait()
    stores[N_CHUNKS - 1].wait()


def kernel(x, mat):
    xf = x.reshape(B_FLAT)
    out_pad = _gather(xf, mat)
    return out_pad.reshape(B_TOKENS, 32, 128)[:, :SEQ, :DIM]


# speedup vs baseline: 3.0494x; 1.5374x over previous
import functools

import jax
import jax.numpy as jnp
from jax import lax
from jax.experimental import pallas as pl
from jax.experimental.pallas import tpu as pltpu
from jax.experimental.pallas import tpu_sc as plsc

NUM_ROWS = 1000000
DIM = 32
B_TOKENS = 16384
SEQ = 26
B_FLAT = B_TOKENS * SEQ
OUT_ROWS = B_TOKENS * 32 * 4
CB = 2048
N_IN_BLOCKS = -(-NUM_ROWS // CB)
STEPS = -(-N_IN_BLOCKS // 4)
IMG_ROWS = STEPS * CB
LIN_ROWS = IMG_ROWS * 4

_info = plsc.get_sparse_core_info()
NC = _info.num_cores
NS = _info.num_subcores
NW = NC * NS
B_PER_W = B_FLAT // NW
CHUNK = 1024
N_CHUNKS = B_PER_W // CHUNK
NBUF = 2
DIV26_MUL = 80660
DIV26_SHIFT = 21


def _tr_body(a0_ref, a1_ref, a2_ref, a3_ref, out_ref):
    eye = jnp.eye(DIM, dtype=jnp.float32)
    dn = (((0,), (0,)), ((), ()))
    parts = [
        lax.dot_general(r[...], eye, dn, preferred_element_type=jnp.float32)
        for r in (a0_ref, a1_ref, a2_ref, a3_ref)
    ]
    out_ref[...] = jnp.concatenate(parts, axis=1)


def _chunk_spec(a):
    return pl.BlockSpec(
        (DIM, CB),
        lambda g, a=a: (0, jnp.minimum(g * 4 + a, N_IN_BLOCKS - 1)))


_transpose = pl.pallas_call(
    _tr_body,
    grid=(STEPS,),
    in_specs=[_chunk_spec(a) for a in range(4)],
    out_specs=pl.BlockSpec((CB, DIM * 4), lambda g: (g, 0)),
    out_shape=jax.ShapeDtypeStruct((IMG_ROWS, DIM * 4), jnp.float32),
)

_mesh = plsc.VectorSubcoreMesh(core_axis_name="c", subcore_axis_name="s")


@functools.partial(
    pl.kernel,
    mesh=_mesh,
    out_type=jax.ShapeDtypeStruct((OUT_ROWS, DIM), jnp.float32),
    compiler_params=pltpu.CompilerParams(use_tc_tiling_on_sc=False),
    scratch_types=[
        pltpu.VMEM((B_PER_W,), jnp.int32),
        pltpu.VMEM((N_CHUNKS, CHUNK), jnp.int32),
        [pltpu.VMEM((CHUNK, DIM), jnp.float32) for _ in range(NBUF)],
        pltpu.SemaphoreType.DMA,
        pltpu.SemaphoreType.DMA,
    ],
)
def _gather(idx_hbm, table_hbm, out_hbm, idx_v, pos_v, rows, gsem, osem):
    wid = lax.axis_index("s") * NC + lax.axis_index("c")
    base = wid * B_PER_W
    tbase = wid * (B_PER_W // SEQ)

    pltpu.sync_copy(idx_hbm.at[pl.ds(base, B_PER_W)], idx_v)

    iota16 = lax.iota(jnp.int32, 16)

    def pbody(mm, carry):
        c = mm // (CHUNK // 16)
        m = mm % (CHUNK // 16)
        off = c * CHUNK + m * 16
        i = idx_v[pl.ds(off, 16)]
        b = lax.shift_right_logical(i, 11)
        j = lax.bitwise_and(i, CB - 1)
        g = lax.shift_right_logical(b, 2)
        idx_v[pl.ds(off, 16)] = (g * CB + j) * 4 + lax.bitwise_and(b, 3)
        jl = iota16 + off
        t = lax.shift_right_logical(jl * DIV26_MUL, DIV26_SHIFT)
        s = jl - t * SEQ
        pos_v[c, pl.ds(m * 16, 16)] = ((tbase + t) * 32 + s) * 4
        return carry

    lax.fori_loop(0, N_CHUNKS * (CHUNK // 16), pbody, 0)

    def start_gather(c):
        return pltpu.async_copy(
            table_hbm.at[idx_v.at[pl.ds(c * CHUNK, CHUNK)]],
            rows[c % NBUF], gsem)

    def start_store(c):
        return pltpu.async_copy(
            rows[c % NBUF], out_hbm.at[pos_v.at[c]], osem)

    gathers = [start_gather(0)]
    stores = []
    for c in range(N_CHUNKS):
        if c + 1 < N_CHUNKS:
            if c >= 1:
                stores[c - 1].wait()
            gathers.append(start_gather(c + 1))
        gathers[c].wait()
        stores.append(start_store(c))
    stores[N_CHUNKS - 2].wait()
    stores[N_CHUNKS - 1].wait()


def kernel(x, mat):
    xf = x.reshape(B_FLAT)
    matT = mat.T
    mat4 = _transpose(matT, matT, matT, matT)
    mat_lin = mat4.reshape(LIN_ROWS, DIM)
    out_pad = _gather(xf, mat_lin)
    return out_pad.reshape(B_TOKENS, 32, 128)[:, :SEQ, :DIM]
